# baseline (speedup 1.0000x reference)
import functools

import jax
import jax.numpy as jnp
from jax import lax
from jax.experimental import pallas as pl
from jax.experimental.pallas import tpu as pltpu
from jax.experimental.pallas import tpu_sc as plsc

B = 4096
L = 200
D = 64
SCALE = float(D) ** 0.5
PAD = 0

_info = plsc.get_sparse_core_info()
NC = _info.num_cores
NS = _info.num_subcores
NW = NC * NS
SEQ_PER_W = B // NW
SPLIT = 104
NBUF = 2
LANES = 16


def _sc_embed_body(log_hbm, item_hbm, pos_hbm, out_hbm,
                   idx_v, pos_v, rows0, rows1, obuf0, obuf1,
                   gsem0, gsem1, wsem0, wsem1):
    rows = [rows0, rows1]
    obuf = [obuf0, obuf1]
    gsem = [gsem0, gsem1]
    wsem = [wsem0, wsem1]

    wid = lax.axis_index("s") * NC + lax.axis_index("c")
    base = wid * SEQ_PER_W

    pltpu.sync_copy(log_hbm.at[pl.ds(base * L, SEQ_PER_W * L)], idx_v)
    pltpu.sync_copy(pos_hbm, pos_v)

    def start_gather(g, b):
        off = g * L
        pltpu.async_copy(item_hbm.at[idx_v.at[pl.ds(off, SPLIT)]],
                         rows[b].at[pl.ds(0, SPLIT)], gsem[b])
        pltpu.async_copy(item_hbm.at[idx_v.at[pl.ds(off + SPLIT, L - SPLIT)]],
                         rows[b].at[pl.ds(SPLIT, L - SPLIT)], gsem[b])

    def drain_gather(b):
        pltpu.make_async_copy(item_hbm.at[pl.ds(0, L)], rows[b], gsem[b]).wait()

    def drain_write(b):
        pltpu.make_async_copy(obuf[b], out_hbm.at[base], wsem[b]).wait()

    def compute(b):
        def row_body(i, carry):
            for j in range(D // LANES):
                sl = pl.ds(j * LANES, LANES)
                obuf[b][i, sl] = rows[b][i, sl] * SCALE + pos_v[i, sl]
            return carry
        lax.fori_loop(0, L, row_body, 0, unroll=2)

    for b in range(NBUF):
        start_gather(b, b)

    def outer(o, carry):
        for b in range(NBUF):
            g = o * NBUF + b
            drain_gather(b)

            @pl.when(o > 0)
            def _():
                drain_write(b)

            compute(b)
            pltpu.async_copy(obuf[b], out_hbm.at[base + g], wsem[b])

            @pl.when(g + NBUF < SEQ_PER_W)
            def _():
                start_gather(g + NBUF, b)
        return carry

    lax.fori_loop(0, SEQ_PER_W // NBUF, outer, 0)

    for b in range(NBUF):
        drain_write(b)


@jax.jit
def _sc_embed(log_flat, item_emb, pos_emb):
    kern = functools.partial(
        pl.kernel,
        out_type=jax.ShapeDtypeStruct((B, L, D), jnp.float32),
        mesh=plsc.VectorSubcoreMesh(core_axis_name="c", subcore_axis_name="s"),
        compiler_params=pltpu.CompilerParams(use_tc_tiling_on_sc=False),
        scratch_types=[
            pltpu.VMEM((SEQ_PER_W * L,), jnp.int32),
            pltpu.VMEM((L, D), jnp.float32),
            pltpu.VMEM((L, D), jnp.float32),
            pltpu.VMEM((L, D), jnp.float32),
            pltpu.VMEM((L, D), jnp.float32),
            pltpu.VMEM((L, D), jnp.float32),
            pltpu.SemaphoreType.DMA,
            pltpu.SemaphoreType.DMA,
            pltpu.SemaphoreType.DMA,
            pltpu.SemaphoreType.DMA,
        ],
    )(_sc_embed_body)
    return kern(log_flat, item_emb, pos_emb)


def _mask_body(seq_ref, mask_ref):
    mask_ref[...] = seq_ref[...] == PAD


@jax.jit
def _tc_mask(log_seqs):
    return pl.pallas_call(
        _mask_body,
        out_shape=jax.ShapeDtypeStruct((B, L), jnp.bool_),
    )(log_seqs)


def kernel(log_seqs, item_emb, pos_emb):
    log_seqs = log_seqs.astype(jnp.int32)
    seqs = _sc_embed(log_seqs.reshape(-1), item_emb, pos_emb)
    mask = _tc_mask(log_seqs)
    return seqs, mask


# speedup vs baseline: 1.3588x; 1.3588x over previous
import functools

import jax
import jax.numpy as jnp
from jax import lax
from jax.experimental import pallas as pl
from jax.experimental.pallas import tpu as pltpu
from jax.experimental.pallas import tpu_sc as plsc

B = 4096
L = 200
D = 64
SCALE = float(D) ** 0.5
PAD = 0

_info = plsc.get_sparse_core_info()
NC = _info.num_cores
NS = _info.num_subcores
NW = NC * NS
SEQ_PER_W = B // NW
SPLIT = 104
NBUF = 3
LANES = 16


def _sc_embed_body(log_hbm, item_hbm, pos_hbm, out_hbm,
                   idx_v, pos_v, rows0, rows1, rows2, obuf0, obuf1, obuf2,
                   gsem0, gsem1, gsem2, wsem0, wsem1, wsem2):
    rows = [rows0, rows1, rows2]
    obuf = [obuf0, obuf1, obuf2]
    gsem = [gsem0, gsem1, gsem2]
    wsem = [wsem0, wsem1, wsem2]

    wid = lax.axis_index("s") * NC + lax.axis_index("c")
    base = wid * SEQ_PER_W

    pltpu.sync_copy(log_hbm.at[pl.ds(base, SEQ_PER_W)], idx_v)
    pltpu.sync_copy(pos_hbm, pos_v)

    def start_gather(g, b):
        pltpu.async_copy(item_hbm.at[idx_v.at[g, pl.ds(0, SPLIT)]],
                         rows[b].at[pl.ds(0, SPLIT)], gsem[b])
        pltpu.async_copy(item_hbm.at[idx_v.at[g, pl.ds(SPLIT, L - SPLIT)]],
                         rows[b].at[pl.ds(SPLIT, L - SPLIT)], gsem[b])

    def drain_gather(b):
        pltpu.make_async_copy(item_hbm.at[pl.ds(0, L)], rows[b], gsem[b]).wait()

    def drain_write(b):
        pltpu.make_async_copy(obuf[b], out_hbm.at[base], wsem[b]).wait()

    def compute(b):
        def row_body(i):
            for j in range(D // LANES):
                sl = pl.ds(j * LANES, LANES)
                obuf[b][i, sl] = rows[b][i, sl] * SCALE + pos_v[i, sl]
        plsc.parallel_loop(0, L, 1, unroll=8)(row_body)

    for b in range(NBUF):
        start_gather(b, b)

    def outer(o, carry):
        for b in range(NBUF):
            g = o * NBUF + b
            drain_gather(b)

            @pl.when(o > 0)
            def _():
                drain_write(b)

            compute(b)
            pltpu.async_copy(obuf[b], out_hbm.at[base + g], wsem[b])

            @pl.when(g + NBUF < SEQ_PER_W)
            def _():
                start_gather(g + NBUF, b)
        return carry

    lax.fori_loop(0, SEQ_PER_W // NBUF, outer, 0)

    REM = SEQ_PER_W % NBUF
    for t in range(REM):
        g = SEQ_PER_W - REM + t
        b = g % NBUF
        drain_gather(b)
        drain_write(b)
        compute(b)
        pltpu.async_copy(obuf[b], out_hbm.at[base + g], wsem[b])

    for b in range(NBUF):
        drain_write(b)


@jax.jit
def _sc_embed(log_seqs, item_emb, pos_emb):
    kern = functools.partial(
        pl.kernel,
        out_type=jax.ShapeDtypeStruct((B, L, D), jnp.float32),
        mesh=plsc.VectorSubcoreMesh(core_axis_name="c", subcore_axis_name="s"),
        compiler_params=pltpu.CompilerParams(use_tc_tiling_on_sc=False),
        scratch_types=[
            pltpu.VMEM((SEQ_PER_W, L), jnp.int32),
            pltpu.VMEM((L, D), jnp.float32),
            pltpu.VMEM((L, D), jnp.float32),
            pltpu.VMEM((L, D), jnp.float32),
            pltpu.VMEM((L, D), jnp.float32),
            pltpu.VMEM((L, D), jnp.float32),
            pltpu.VMEM((L, D), jnp.float32),
            pltpu.VMEM((L, D), jnp.float32),
            pltpu.SemaphoreType.DMA,
            pltpu.SemaphoreType.DMA,
            pltpu.SemaphoreType.DMA,
            pltpu.SemaphoreType.DMA,
            pltpu.SemaphoreType.DMA,
            pltpu.SemaphoreType.DMA,
        ],
    )(_sc_embed_body)
    return kern(log_seqs, item_emb, pos_emb)


def _mask_body(seq_ref, mask_ref):
    mask_ref[...] = seq_ref[...] == PAD


@jax.jit
def _tc_mask(log_seqs):
    return pl.pallas_call(
        _mask_body,
        out_shape=jax.ShapeDtypeStruct((B, L), jnp.bool_),
    )(log_seqs)


def kernel(log_seqs, item_emb, pos_emb):
    log_seqs = log_seqs.astype(jnp.int32)
    seqs = _sc_embed(log_seqs, item_emb, pos_emb)
    mask = _tc_mask(log_seqs)
    return seqs, mask
